# baseline (speedup 1.0000x reference)
import jax
import jax.numpy as jnp
from jax import lax
from jax.experimental import pallas as pl
from jax.experimental.pallas import tpu as pltpu
from jax.experimental.pallas import tpu_sc as plsc

_M, _N = 65536, 1024
_NQ = 64
_NW = 32
_RPW = _M // _NW
_CH = 16
_NBUF = 4
_NCH = _RPW // _CH
_NGRP = _NCH // _NBUF


def _fill(buf, chanv, chanbase):
    for t in range(_CH // 16):
        c = chanv[pl.ds(chanbase + t * 16, 16)]
        for j in range(16):
            v = jnp.full((16,), c[j], jnp.float32)
            r = t * 16 + j
            for k in range(_NQ // 16):
                buf[r, pl.ds(k * 16, 16)] = v


def _sc_body(data_hbm, chan_hbm, out_hbm, chanv,
             buf0, buf1, buf2, buf3,
             in0, in1, in2, in3, ou0, ou1, ou2, ou3):
    bufs = (buf0, buf1, buf2, buf3)
    ins = (in0, in1, in2, in3)
    outs = (ou0, ou1, ou2, ou3)
    c = lax.axis_index("c")
    s = lax.axis_index("s")
    wid = s * 2 + c
    base = wid * _RPW
    pltpu.sync_copy(chan_hbm.at[pl.ds(base, _RPW)], chanv)

    def in_copy(chunk, buf, sem):
        return pltpu.async_copy(
            data_hbm.at[pl.ds(base + chunk * _CH, _CH)], buf, sem)

    def in_wait(chunk, buf, sem):
        pltpu.make_async_copy(
            data_hbm.at[pl.ds(base + chunk * _CH, _CH)], buf, sem).wait()

    def out_copy(chunk, buf, sem):
        return pltpu.async_copy(
            buf, out_hbm.at[pl.ds(base + chunk * _CH, _CH)], sem)

    def out_wait(chunk, buf, sem):
        pltpu.make_async_copy(
            buf, out_hbm.at[pl.ds(base + chunk * _CH, _CH)], sem).wait()

    for b in range(_NBUF):
        in_copy(b, bufs[b], ins[b])

    def group(g, carry):
        ch0 = g * _NBUF
        for b in range(_NBUF):
            ch = ch0 + b
            in_wait(ch, bufs[b], ins[b])
            _fill(bufs[b], chanv, ch * _CH)
            out_copy(ch, bufs[b], outs[b])
            nb = (b + 1) % _NBUF
            if b < _NBUF - 1:
                @pl.when(g > 0)
                def _p():
                    out_wait(ch - (_NBUF - 1), bufs[nb], outs[nb])
                    in_copy(ch + 1, bufs[nb], ins[nb])
            else:
                @pl.when(g < _NGRP - 1)
                def _p():
                    out_wait(ch - (_NBUF - 1), bufs[nb], outs[nb])
                    in_copy(ch + 1, bufs[nb], ins[nb])
        return carry

    lax.fori_loop(0, _NGRP, group, 0)
    for tail in range(_NCH - _NBUF, _NCH):
        out_wait(tail, bufs[tail % _NBUF], outs[tail % _NBUF])


def kernel(data, query, channel_index):
    del query
    chanf = channel_index.astype(data.dtype)
    mesh = plsc.VectorSubcoreMesh(core_axis_name="c", subcore_axis_name="s")
    f = pl.kernel(
        _sc_body,
        out_type=jax.ShapeDtypeStruct((_M, _N), data.dtype),
        mesh=mesh,
        scratch_types=(
            [pltpu.VMEM((_RPW,), jnp.float32)]
            + [pltpu.VMEM((_CH, _N), jnp.float32) for _ in range(_NBUF)]
            + [pltpu.SemaphoreType.DMA for _ in range(2 * _NBUF)]
        ),
    )
    return f(data, chanf)


# speedup vs baseline: 1.2384x; 1.2384x over previous
import jax
import jax.numpy as jnp
from jax import lax
from jax.experimental import pallas as pl
from jax.experimental.pallas import tpu as pltpu
from jax.experimental.pallas import tpu_sc as plsc

_M, _N = 65536, 1024
_NQ = 64
_NW = 32
_RPW = _M // _NW
_CH = 16
_NBUF = 6
_NCH = _RPW // _CH
_NGRP = (_NCH - 2) // _NBUF
_TAIL = _NCH - _NGRP * _NBUF


def _fill(buf, chanv, chanbase):
    for t in range(_CH // 16):
        c = chanv[pl.ds(chanbase + t * 16, 16)]
        for j in range(16):
            v = jnp.full((16,), c[j], jnp.float32)
            r = t * 16 + j
            for k in range(_NQ // 16):
                buf[r, pl.ds(k * 16, 16)] = v


def _sc_body(data_hbm, chan_hbm, out_hbm, chanv,
             buf0, buf1, buf2, buf3, buf4, buf5,
             in0, in1, in2, in3, in4, in5,
             ou0, ou1, ou2, ou3, ou4, ou5):
    bufs = (buf0, buf1, buf2, buf3, buf4, buf5)
    ins = (in0, in1, in2, in3, in4, in5)
    outs = (ou0, ou1, ou2, ou3, ou4, ou5)
    c = lax.axis_index("c")
    s = lax.axis_index("s")
    wid = s * 2 + c
    base = wid * _RPW
    pltpu.sync_copy(chan_hbm.at[pl.ds(base, _RPW)], chanv)

    def in_copy(chunk, buf, sem):
        return pltpu.async_copy(
            data_hbm.at[pl.ds(base + chunk * _CH, _CH)], buf, sem)

    def in_wait(chunk, buf, sem):
        pltpu.make_async_copy(
            data_hbm.at[pl.ds(base + chunk * _CH, _CH)], buf, sem).wait()

    def out_copy(chunk, buf, sem):
        return pltpu.async_copy(
            buf, out_hbm.at[pl.ds(base + chunk * _CH, _CH)], sem)

    for b in range(_NBUF):
        in_copy(b, bufs[b], ins[b])

    def group(g, carry):
        ch0 = g * _NBUF
        handles = []
        for b in range(_NBUF):
            ch = ch0 + b
            in_wait(ch, bufs[b], ins[b])
            _fill(bufs[b], chanv, ch * _CH)
            handles.append(out_copy(ch, bufs[b], outs[b]))

        @pl.when(g < _NGRP - 1)
        def _prefetch():
            for b in range(_NBUF):
                handles[b].wait()
                in_copy(ch0 + _NBUF + b, bufs[b], ins[b])

        @pl.when(g == _NGRP - 1)
        def _drain():
            for b in range(_NBUF):
                handles[b].wait()
                if b < _TAIL:
                    in_copy(_NGRP * _NBUF + b, bufs[b], ins[b])

        return carry

    lax.fori_loop(0, _NGRP, group, 0)
    for t in range(_TAIL):
        ch = _NGRP * _NBUF + t
        in_wait(ch, bufs[t], ins[t])
        _fill(bufs[t], chanv, ch * _CH)
        out_copy(ch, bufs[t], outs[t]).wait()


def kernel(data, query, channel_index):
    del query
    chanf = channel_index.astype(data.dtype)
    mesh = plsc.VectorSubcoreMesh(core_axis_name="c", subcore_axis_name="s")
    f = pl.kernel(
        _sc_body,
        out_type=jax.ShapeDtypeStruct((_M, _N), data.dtype),
        mesh=mesh,
        scratch_types=(
            [pltpu.VMEM((_RPW,), jnp.float32)]
            + [pltpu.VMEM((_CH, _N), jnp.float32) for _ in range(_NBUF)]
            + [pltpu.SemaphoreType.DMA for _ in range(2 * _NBUF)]
        ),
    )
    return f(data, chanf)


# speedup vs baseline: 1.2471x; 1.0070x over previous
import jax
import jax.numpy as jnp
from jax import lax
from jax.experimental import pallas as pl
from jax.experimental.pallas import tpu as pltpu
from jax.experimental.pallas import tpu_sc as plsc

_M, _N = 65536, 1024
_NQ = 64
_NW = 32
_RPW = _M // _NW
_CH = 8
_NBUF = 4
_NCH = _RPW // _CH
_NGRP = _NCH // _NBUF


def _fill(buf, chanv, chanbase):
    c = chanv[pl.ds(chanbase, 16)]
    for j in range(_CH):
        v = jnp.full((16,), c[j], jnp.float32)
        for k in range(_NQ // 16):
            buf[j, pl.ds(k * 16, 16)] = v


def _sc_body(data_hbm, chan_hbm, out_hbm, chanv,
             buf0, buf1, buf2, buf3,
             in0, in1, in2, in3, ou0, ou1, ou2, ou3):
    bufs = (buf0, buf1, buf2, buf3)
    ins = (in0, in1, in2, in3)
    outs = (ou0, ou1, ou2, ou3)
    c = lax.axis_index("c")
    s = lax.axis_index("s")
    wid = s * 2 + c
    base = wid * _RPW
    pltpu.sync_copy(chan_hbm.at[pl.ds(base, _RPW)], chanv.at[pl.ds(0, _RPW)])

    def in_copy(chunk, buf, sem):
        return pltpu.async_copy(
            data_hbm.at[pl.ds(base + chunk * _CH, _CH)], buf, sem)

    def in_wait(chunk, buf, sem):
        pltpu.make_async_copy(
            data_hbm.at[pl.ds(base + chunk * _CH, _CH)], buf, sem).wait()

    def out_copy(chunk, buf, sem):
        return pltpu.async_copy(
            buf, out_hbm.at[pl.ds(base + chunk * _CH, _CH)], sem)

    for b in range(_NBUF):
        in_copy(b, bufs[b], ins[b])

    def group(g, carry):
        ch0 = g * _NBUF
        handles = []
        for b in range(_NBUF):
            ch = ch0 + b
            in_wait(ch, bufs[b], ins[b])
            _fill(bufs[b], chanv, ch * _CH)
            handles.append(out_copy(ch, bufs[b], outs[b]))

        @pl.when(g < _NGRP - 1)
        def _prefetch():
            for b in range(_NBUF):
                handles[b].wait()
                in_copy(ch0 + _NBUF + b, bufs[b], ins[b])

        @pl.when(g == _NGRP - 1)
        def _drain():
            for b in range(_NBUF):
                handles[b].wait()

        return carry

    lax.fori_loop(0, _NGRP, group, 0)


def kernel(data, query, channel_index):
    del query
    chanf = channel_index.astype(data.dtype)
    mesh = plsc.VectorSubcoreMesh(core_axis_name="c", subcore_axis_name="s")
    f = pl.kernel(
        _sc_body,
        out_type=jax.ShapeDtypeStruct((_M, _N), data.dtype),
        mesh=mesh,
        scratch_types=(
            [pltpu.VMEM((_RPW + 16,), jnp.float32)]
            + [pltpu.VMEM((_CH, _N), jnp.float32) for _ in range(_NBUF)]
            + [pltpu.SemaphoreType.DMA for _ in range(2 * _NBUF)]
        ),
    )
    return f(data, chanf)


# speedup vs baseline: 1.2498x; 1.0022x over previous
import jax
import jax.numpy as jnp
from jax import lax
from jax.experimental import pallas as pl
from jax.experimental.pallas import tpu as pltpu
from jax.experimental.pallas import tpu_sc as plsc

_M, _N = 65536, 1024
_NQ = 64
_NW = 32
_RPW = _M // _NW
_CH = 16
_NBUF = 4
_NCH = _RPW // _CH
_NGRP = _NCH // _NBUF


def _fill(buf, chanv, chanbase):
    for t in range(_CH // 16):
        c = chanv[pl.ds(chanbase + t * 16, 16)]
        for j in range(16):
            v = jnp.full((16,), c[j], jnp.float32)
            r = t * 16 + j
            for k in range(_NQ // 16):
                buf[r, pl.ds(k * 16, 16)] = v


def _sc_body(data_hbm, chan_hbm, out_hbm, chanv,
             buf0, buf1, buf2, buf3,
             in0, in1, in2, in3, ou0, ou1, ou2, ou3):
    bufs = (buf0, buf1, buf2, buf3)
    ins = (in0, in1, in2, in3)
    outs = (ou0, ou1, ou2, ou3)
    c = lax.axis_index("c")
    s = lax.axis_index("s")
    wid = s * 2 + c
    base = wid * _RPW
    pltpu.sync_copy(chan_hbm.at[pl.ds(base, _RPW)], chanv)

    def in_copy(chunk, buf, sem):
        return pltpu.async_copy(
            data_hbm.at[pl.ds(base + chunk * _CH, _CH)], buf, sem)

    def in_wait(chunk, buf, sem):
        pltpu.make_async_copy(
            data_hbm.at[pl.ds(base + chunk * _CH, _CH)], buf, sem).wait()

    def out_copy(chunk, buf, sem):
        return pltpu.async_copy(
            buf, out_hbm.at[pl.ds(base + chunk * _CH, _CH)], sem)

    for b in range(_NBUF):
        in_copy(b, bufs[b], ins[b])

    def group(g, carry):
        ch0 = g * _NBUF
        handles = []
        for b in range(_NBUF):
            ch = ch0 + b
            in_wait(ch, bufs[b], ins[b])
            _fill(bufs[b], chanv, ch * _CH)
            handles.append(out_copy(ch, bufs[b], outs[b]))

        @pl.when(g < _NGRP - 1)
        def _prefetch():
            for b in range(_NBUF):
                handles[b].wait()
                in_copy(ch0 + _NBUF + b, bufs[b], ins[b])

        @pl.when(g == _NGRP - 1)
        def _drain():
            for b in range(_NBUF):
                handles[b].wait()

        return carry

    lax.fori_loop(0, _NGRP, group, 0)


def kernel(data, query, channel_index):
    del query
    chanf = channel_index.astype(data.dtype)
    mesh = plsc.VectorSubcoreMesh(core_axis_name="c", subcore_axis_name="s")
    f = pl.kernel(
        _sc_body,
        out_type=jax.ShapeDtypeStruct((_M, _N), data.dtype),
        mesh=mesh,
        scratch_types=(
            [pltpu.VMEM((_RPW,), jnp.float32)]
            + [pltpu.VMEM((_CH, _N), jnp.float32) for _ in range(_NBUF)]
            + [pltpu.SemaphoreType.DMA for _ in range(2 * _NBUF)]
        ),
    )
    return f(data, chanf)
